# Initial kernel scaffold; baseline (speedup 1.0000x reference)
import jax
import jax.numpy as jnp
from jax.experimental import pallas as pl


def kernel(xyz, W_planes, W_shapes, b_shapes):
    raise NotImplementedError("write your pallas kernel here")


# speedup vs baseline: 8.1545x; 8.1545x over previous
import jax
import jax.numpy as jnp
from jax.experimental import pallas as pl
from jax.experimental.pallas import tpu as pltpu

_K = 16


def _ls_kernel(q_ref, pt_ref, p_ref, wp_ref, ws_ref, b_ref, shp_ref, idx_ref):
    q = q_ref[0]
    pt = pt_ref[0]
    p = p_ref[0]
    n = pt.shape[1]
    qb = q.shape[0]

    q2 = jnp.sum(q * q, axis=1, keepdims=True)
    p2 = jnp.sum(pt * pt, axis=0, keepdims=True)
    qp = jax.lax.dot_general(q, pt, (((1,), (0,)), ((), ())),
                             preferred_element_type=jnp.float32)
    dist = q2 + p2 - 2.0 * qp

    iota_n = jax.lax.broadcasted_iota(jnp.int32, (qb, n), 1)
    iota_k = jax.lax.broadcasted_iota(jnp.int32, (qb, _K), 1)
    wp = wp_ref[...]

    def body(t, carry):
        d, acc, idxs = carry
        m = jnp.min(d, axis=1, keepdims=True)
        arg = jnp.min(jnp.where(d <= m, iota_n, n), axis=1, keepdims=True)
        onehot = jnp.where(iota_n == arg, 1.0, 0.0)
        d = d + onehot * 1e30
        idxs = jnp.where(iota_k == t, arg, idxs)
        coords = jax.lax.dot_general(onehot, p, (((1,), (0,)), ((), ())),
                                     preferred_element_type=jnp.float32)
        v = coords - q
        nrm = jnp.sqrt(jnp.sum(v * v, axis=1, keepdims=True)) + 1e-8
        pr = jax.lax.dot_general(v, wp, (((1,), (1,)), ((), ())),
                                 preferred_element_type=jnp.float32)
        score = pr * jnp.abs(pr) / nrm
        acc = jnp.where(t > 0, jnp.maximum(acc, score), acc)
        return d, acc, idxs

    acc0 = jnp.full((qb, 64), -jnp.inf, dtype=jnp.float32)
    idx0 = jnp.zeros((qb, _K), dtype=jnp.int32)
    _, acc, idxs = jax.lax.fori_loop(0, _K, body, (dist, acc0, idx0))

    ws = ws_ref[...]
    out = jax.lax.dot_general(acc, ws, (((1,), (1,)), ((), ())),
                              preferred_element_type=jnp.float32) + b_ref[...]
    shp_ref[0] = jnp.maximum(out, 0.0)
    idx_ref[0] = idxs


def kernel(xyz, W_planes, W_shapes, b_shapes):
    B, N, _ = xyz.shape
    qb = min(256, N)
    xyz_t = jnp.transpose(xyz, (0, 2, 1))
    b2 = b_shapes.reshape(1, -1)
    nshapes = W_shapes.shape[0]

    shp, idx = pl.pallas_call(
        _ls_kernel,
        grid=(B, N // qb),
        in_specs=[
            pl.BlockSpec((1, qb, 3), lambda b, i: (b, i, 0)),
            pl.BlockSpec((1, 3, N), lambda b, i: (b, 0, 0)),
            pl.BlockSpec((1, N, 3), lambda b, i: (b, 0, 0)),
            pl.BlockSpec(W_planes.shape, lambda b, i: (0, 0)),
            pl.BlockSpec(W_shapes.shape, lambda b, i: (0, 0)),
            pl.BlockSpec((1, nshapes), lambda b, i: (0, 0)),
        ],
        out_specs=[
            pl.BlockSpec((1, qb, nshapes), lambda b, i: (b, i, 0)),
            pl.BlockSpec((1, qb, _K), lambda b, i: (b, i, 0)),
        ],
        out_shape=[
            jax.ShapeDtypeStruct((B, N, nshapes), jnp.float32),
            jax.ShapeDtypeStruct((B, N, _K), jnp.int32),
        ],
        compiler_params=pltpu.CompilerParams(
            dimension_semantics=("parallel", "parallel")),
    )(xyz, xyz_t, xyz, W_planes, W_shapes, b2)

    shapes = jnp.transpose(shp, (0, 2, 1))
    return (shapes, xyz, idx)


# speedup vs baseline: 8.4523x; 1.0365x over previous
import jax
import jax.numpy as jnp
from jax.experimental import pallas as pl
from jax.experimental.pallas import tpu as pltpu

_K = 16


def _ls_kernel(q_ref, pt_ref, p_ref, wp_ref, ws_ref, b_ref, shp_ref, idx_ref):
    q = q_ref[0]
    pt = pt_ref[0]
    p = p_ref[0]
    n = pt.shape[1]
    qb = q.shape[0]

    q2 = jnp.sum(q * q, axis=1, keepdims=True)
    p2 = jnp.sum(pt * pt, axis=0, keepdims=True)
    qp = jax.lax.dot_general(q, pt, (((1,), (0,)), ((), ())),
                             preferred_element_type=jnp.float32)
    dist = q2 + p2 - 2.0 * qp

    iota_n = jax.lax.broadcasted_iota(jnp.int32, (qb, n), 1)
    iota_k = jax.lax.broadcasted_iota(jnp.int32, (qb, _K), 1)
    wp = wp_ref[...]

    def body(t, carry):
        d, acc, idxs = carry
        arg = jnp.argmin(d, axis=1, keepdims=True).astype(jnp.int32)
        onehot = jnp.where(iota_n == arg, 1.0, 0.0)
        d = d + onehot * 1e30
        idxs = jnp.where(iota_k == t, arg, idxs)
        coords = jax.lax.dot_general(onehot, p, (((1,), (0,)), ((), ())),
                                     preferred_element_type=jnp.float32)
        v = coords - q
        nrm = jnp.sqrt(jnp.sum(v * v, axis=1, keepdims=True)) + 1e-8
        pr = jax.lax.dot_general(v, wp, (((1,), (1,)), ((), ())),
                                 preferred_element_type=jnp.float32)
        score = pr * jnp.abs(pr) / nrm
        acc = jnp.where(t > 0, jnp.maximum(acc, score), acc)
        return d, acc, idxs

    acc0 = jnp.full((qb, 64), -jnp.inf, dtype=jnp.float32)
    idx0 = jnp.zeros((qb, _K), dtype=jnp.int32)
    _, acc, idxs = jax.lax.fori_loop(0, _K, body, (dist, acc0, idx0))

    ws = ws_ref[...]
    out = jax.lax.dot_general(acc, ws, (((1,), (1,)), ((), ())),
                              preferred_element_type=jnp.float32) + b_ref[...]
    shp_ref[0] = jnp.maximum(out, 0.0)
    idx_ref[0] = idxs


def kernel(xyz, W_planes, W_shapes, b_shapes):
    B, N, _ = xyz.shape
    qb = min(256, N)
    xyz_t = jnp.transpose(xyz, (0, 2, 1))
    b2 = b_shapes.reshape(1, -1)
    nshapes = W_shapes.shape[0]

    shp, idx = pl.pallas_call(
        _ls_kernel,
        grid=(B, N // qb),
        in_specs=[
            pl.BlockSpec((1, qb, 3), lambda b, i: (b, i, 0)),
            pl.BlockSpec((1, 3, N), lambda b, i: (b, 0, 0)),
            pl.BlockSpec((1, N, 3), lambda b, i: (b, 0, 0)),
            pl.BlockSpec(W_planes.shape, lambda b, i: (0, 0)),
            pl.BlockSpec(W_shapes.shape, lambda b, i: (0, 0)),
            pl.BlockSpec((1, nshapes), lambda b, i: (0, 0)),
        ],
        out_specs=[
            pl.BlockSpec((1, qb, nshapes), lambda b, i: (b, i, 0)),
            pl.BlockSpec((1, qb, _K), lambda b, i: (b, i, 0)),
        ],
        out_shape=[
            jax.ShapeDtypeStruct((B, N, nshapes), jnp.float32),
            jax.ShapeDtypeStruct((B, N, _K), jnp.int32),
        ],
        compiler_params=pltpu.CompilerParams(
            dimension_semantics=("parallel", "parallel")),
    )(xyz, xyz_t, xyz, W_planes, W_shapes, b2)

    shapes = jnp.transpose(shp, (0, 2, 1))
    return (shapes, xyz, idx)
